# baseline (speedup 1.0000x reference)
import functools

import jax
import jax.numpy as jnp
from jax import lax
from jax.experimental import pallas as pl
from jax.experimental.pallas import tpu as pltpu
from jax.experimental.pallas import tpu_sc as plsc

SPAD = 256
NC, NS = 2, 16
NW = NC * NS


def _attn_call(dh, enc, ids, wa, wb, ba, wv, wgh, wgc, bg):
    B, S, H = enc.shape
    BB = 8
    V = None

    def body(dh_ref, enc_ref, ids_ref, wa_ref, wb_ref, ba_ref, wv_ref,
             wgh_ref, wgc_ref, bg_ref, vcb_ref,
             cw_ref, gate_ref, upd_ref, idsf_ref):
        pid = pl.program_id(0)
        dhb = dh_ref[...]
        encb = enc_ref[...]
        dpart = jnp.dot(dhb, wa_ref[...], preferred_element_type=jnp.float32)
        dpart = dpart + ba_ref[...]
        e = jnp.dot(encb.reshape(BB * S, H), wb_ref[...],
                    preferred_element_type=jnp.float32)
        e = jnp.tanh(e.reshape(BB, S, H) + dpart[:, None, :])
        sc = jnp.sum(e * wv_ref[...][None, :, :], axis=2)
        m = jnp.max(sc, axis=1, keepdims=True)
        ex = jnp.exp(sc - m)
        w = ex / jnp.sum(ex, axis=1, keepdims=True)
        cw_ref[...] = w
        ctx = jnp.sum(w[:, :, None] * encb, axis=1)
        g = jnp.dot(dhb, wgh_ref[...], preferred_element_type=jnp.float32)
        g = g + jnp.dot(ctx, wgc_ref[...], preferred_element_type=jnp.float32)
        g = jax.nn.sigmoid(g + bg_ref[...])
        gate_ref[...] = g
        idsb = ids_ref[...]
        ids_pad = jnp.concatenate(
            [idsb, jnp.zeros((BB, SPAD - S), jnp.int32)], axis=1)
        w_pad = jnp.concatenate(
            [w, jnp.zeros((BB, SPAD - S), jnp.float32)], axis=1)
        eq = (ids_pad[:, :, None] == ids_pad[:, None, :]).astype(jnp.float32)
        comb = jnp.sum(w_pad[:, :, None] * eq, axis=1)
        upd_ref[...] = comb * g
        row = pid * BB + lax.broadcasted_iota(jnp.int32, (BB, 1), 0)
        idsf_ref[...] = ids_pad + row * vcb_ref[0]

    def run(vocab_size):
        vcb = jnp.full((1,), vocab_size, jnp.int32)
        return pl.pallas_call(
            body,
            grid=(B // BB,),
            in_specs=[
                pl.BlockSpec((BB, H), lambda i: (i, 0)),
                pl.BlockSpec((BB, S, H), lambda i: (i, 0, 0)),
                pl.BlockSpec((BB, S), lambda i: (i, 0)),
                pl.BlockSpec((H, H), lambda i: (0, 0)),
                pl.BlockSpec((H, H), lambda i: (0, 0)),
                pl.BlockSpec((1, H), lambda i: (0, 0)),
                pl.BlockSpec((1, H), lambda i: (0, 0)),
                pl.BlockSpec((H, 1), lambda i: (0, 0)),
                pl.BlockSpec((H, 1), lambda i: (0, 0)),
                pl.BlockSpec((1, 1), lambda i: (0, 0)),
                pl.BlockSpec(memory_space=pltpu.SMEM),
            ],
            out_specs=[
                pl.BlockSpec((BB, S), lambda i: (i, 0)),
                pl.BlockSpec((BB, 1), lambda i: (i, 0)),
                pl.BlockSpec((BB, SPAD), lambda i: (i, 0)),
                pl.BlockSpec((BB, SPAD), lambda i: (i, 0)),
            ],
            out_shape=[
                jax.ShapeDtypeStruct((B, S), jnp.float32),
                jax.ShapeDtypeStruct((B, 1), jnp.float32),
                jax.ShapeDtypeStruct((B, SPAD), jnp.float32),
                jax.ShapeDtypeStruct((B, SPAD), jnp.int32),
            ],
        )(dh, enc, ids, wa, wb, ba, wv, wgh, wgc, bg, vcb)

    return run


def _scale(vocab_dist, gate):
    B, V = vocab_dist.shape
    RB, WV = 128, 8192
    if B < RB:
        RB = B
    nv = pl.cdiv(V, WV)

    def body(g_ref, v_ref, o_ref):
        o_ref[...] = (1.0 - g_ref[...]) * v_ref[...]

    return pl.pallas_call(
        body,
        grid=(B // RB, nv),
        in_specs=[
            pl.BlockSpec((RB, 1), lambda i, j: (i, 0)),
            pl.BlockSpec((RB, WV), lambda i, j: (i, j)),
        ],
        out_specs=pl.BlockSpec((RB, WV), lambda i, j: (i, j)),
        out_shape=jax.ShapeDtypeStruct((B, V), jnp.float32),
    )(gate, vocab_dist)


def _make_sc_rmw(B):
    rows = B // NW
    mesh = plsc.VectorSubcoreMesh(core_axis_name="c", subcore_axis_name="s")

    @functools.partial(
        pl.kernel,
        out_type=(),
        mesh=mesh,
        scratch_types=[
            pltpu.VMEM((2, 128), jnp.int32),
            pltpu.VMEM((2, 128), jnp.float32),
            pltpu.VMEM((2, 128), jnp.float32),
            pltpu.SemaphoreType.DMA,
            pltpu.SemaphoreType.DMA,
        ],
    )
    def sc_rmw(final_ref, idsf_hbm, upd_hbm, idx_v, upd_v, old_v, gsem, ssem):
        wid = lax.axis_index("s") * NC + lax.axis_index("c")
        base = wid * rows

        def body(i, carry):
            r = base + i
            pltpu.sync_copy(idsf_hbm.at[r], idx_v)
            pltpu.sync_copy(upd_hbm.at[r], upd_v)
            g0 = pltpu.async_copy(final_ref.at[idx_v.at[0]], old_v.at[0], gsem)
            g1 = pltpu.async_copy(final_ref.at[idx_v.at[1]], old_v.at[1], gsem)
            g0.wait()
            g1.wait()
            for j in range(2):
                for k in range(8):
                    sl = pl.ds(k * 16, 16)
                    old_v[j, sl] = old_v[j, sl] + upd_v[j, sl]
            s0 = pltpu.async_copy(old_v.at[0], final_ref.at[idx_v.at[0]], ssem)
            s1 = pltpu.async_copy(old_v.at[1], final_ref.at[idx_v.at[1]], ssem)
            s0.wait()
            s1.wait()
            return carry

        lax.fori_loop(0, rows, body, 0)

    return sc_rmw


def kernel(decoder_hidden, encoder_outputs, vocab_dist, input_ids,
           W_att, b_att, W_v, W_gate, b_gate):
    B, S, H = encoder_outputs.shape
    V = vocab_dist.shape[1]
    wa = W_att[:, :H].T
    wb = W_att[:, H:].T
    ba = b_att.reshape(1, H)
    wv = W_v.reshape(1, H)
    wgh = W_gate[:, :H].T
    wgc = W_gate[:, H:].T
    bg = b_gate.reshape(1, 1)
    ids = input_ids.astype(jnp.int32)

    cw, gate, upd, idsf = _attn_call(
        decoder_hidden, encoder_outputs, ids, wa, wb, ba, wv, wgh, wgc, bg
    )(V)
    final0 = _scale(vocab_dist, gate)
    fref = jax.new_ref(final0.reshape(B * V))
    _make_sc_rmw(B)(fref,
                    idsf.reshape(B, 2, 128),
                    upd.reshape(B, 2, 128))
    final = jax.freeze(fref).reshape(B, V)
    return final, cw
